# baseline (speedup 1.0000x reference)
import functools

import jax
import jax.numpy as jnp
from jax import lax
from jax.experimental import pallas as pl
from jax.experimental.pallas import tpu as pltpu
from jax.experimental.pallas import tpu_sc as plsc

N = 32768
H1 = 16
H2 = 32
E = 8
L = 16
NW = 32
NB = N // NW
U = 2
G = NB // (L * U)

_W1_OFF = 0
_B1_OFF = _W1_OFF + H1
_W2_OFF = _B1_OFF + H1
_B2_OFF = _W2_OFF + H1 * H2
_W3_OFF = _B2_OFF + H2
_B3_OFF = _W3_OFF + H2 * E
_WTOT = _B3_OFF + E

_mesh = plsc.VectorSubcoreMesh(core_axis_name="c", subcore_axis_name="s")


@functools.partial(
    pl.kernel,
    mesh=_mesh,
    out_type=jax.ShapeDtypeStruct((N * E,), jnp.float32),
    scratch_types=[
        pltpu.VMEM((NB,), jnp.float32),
        pltpu.VMEM((_WTOT * L,), jnp.float32),
        pltpu.VMEM((NB * E,), jnp.float32),
        pltpu.SemaphoreType.DMA,
        pltpu.SemaphoreType.DMA,
    ],
    compiler_params=pltpu.CompilerParams(needs_layout_passes=False),
)
def _router(x_hbm, w_hbm, out_hbm, x_v, w_v, out_v, sem_x, sem_w):
    wid = lax.axis_index("s") * 2 + lax.axis_index("c")
    base = wid * NB
    cp_x = pltpu.async_copy(x_hbm.at[pl.ds(base, NB)], x_v, sem_x)
    cp_w = pltpu.async_copy(w_hbm, w_v, sem_w)
    cp_x.wait()
    cp_w.wait()

    lane = jnp.arange(L, dtype=jnp.int32)

    def wvec(i):
        return w_v[pl.ds(i * L, L)]

    def body(g, carry):
        g0 = g * U
        xs = [x_v[pl.ds((g0 + u) * L, L)] for u in range(U)]
        h1 = [[None] * H1 for _ in range(U)]
        for j in range(H1):
            w = wvec(_W1_OFF + j)
            b = wvec(_B1_OFF + j)
            for u in range(U):
                h1[u][j] = jnp.maximum(xs[u] * w + b, 0.0)
        h2 = [[None] * H2 for _ in range(U)]
        for k in range(H2):
            b = wvec(_B2_OFF + k)
            w0 = wvec(_W2_OFF + k)
            acc = [h1[u][0] * w0 + b for u in range(U)]
            for j in range(1, H1):
                w = wvec(_W2_OFF + j * H2 + k)
                for u in range(U):
                    acc[u] = h1[u][j] * w + acc[u]
            for u in range(U):
                h2[u][k] = jnp.maximum(acc[u], 0.0)
        logit = [[None] * E for _ in range(U)]
        for e in range(E):
            b = wvec(_B3_OFF + e)
            w0 = wvec(_W3_OFF + e)
            acc = [h2[u][0] * w0 + b for u in range(U)]
            for k in range(1, H2):
                w = wvec(_W3_OFF + k * E + e)
                for u in range(U):
                    acc[u] = h2[u][k] * w + acc[u]
            for u in range(U):
                logit[u][e] = acc[u]
        for u in range(U):
            lg = logit[u]
            m = lg[0]
            for e in range(1, E):
                m = jnp.maximum(m, lg[e])
            ex = [jnp.exp(lg[e] - m) for e in range(E)]
            s = ex[0]
            for e in range(1, E):
                s = s + ex[e]
            inv = 1.0 / s
            flat0 = ((g0 + u) * L + lane) * E
            for e in range(E):
                plsc.store_scatter(out_v, [flat0 + e], ex[e] * inv)
        return carry

    lax.fori_loop(0, G, body, 0)
    pltpu.sync_copy(out_v, out_hbm.at[pl.ds(base * E, NB * E)])


def kernel(snr_estimate, temperature, W1, b1, W2, b2, W3, b3):
    x = snr_estimate.reshape(N)
    inv_t = 1.0 / temperature
    packed = jnp.concatenate([
        W1.reshape(H1), b1,
        W2.reshape(H1 * H2), b2,
        (W3 * inv_t).reshape(H2 * E), b3 * inv_t,
    ])
    replicated = jnp.broadcast_to(packed[:, None], (_WTOT, L)).reshape(_WTOT * L)
    return _router(x, replicated).reshape(N, E)


# speedup vs baseline: 2.4719x; 2.4719x over previous
import jax
import jax.numpy as jnp
from jax import lax
from jax.experimental import pallas as pl
from jax.experimental.pallas import tpu as pltpu

N = 32768
H1 = 16
H2 = 32
E = 8


def _body(x_ref, w1_ref, b1_ref, w2_ref, b2_ref, w3_ref, b3_ref, out_ref):
    x = x_ref[...]
    h1 = jnp.maximum(w1_ref[...] * x + b1_ref[...], 0.0)
    h2 = lax.dot_general(w2_ref[...], h1, (((0,), (0,)), ((), ())),
                         preferred_element_type=jnp.float32)
    h2 = jnp.maximum(h2 + b2_ref[...], 0.0)
    lg = lax.dot_general(w3_ref[...], h2, (((0,), (0,)), ((), ())),
                         preferred_element_type=jnp.float32)
    lg = lg + b3_ref[...]
    m = jnp.max(lg, axis=0, keepdims=True)
    p = jnp.exp(lg - m)
    s = jnp.sum(p, axis=0, keepdims=True)
    out_ref[...] = (p / s).T


def kernel(snr_estimate, temperature, W1, b1, W2, b2, W3, b3):
    inv_t = 1.0 / temperature
    return pl.pallas_call(
        _body,
        out_shape=jax.ShapeDtypeStruct((N, E), jnp.float32),
    )(
        snr_estimate.reshape(1, N),
        W1.reshape(H1, 1), b1.reshape(H1, 1),
        W2, b2.reshape(H2, 1),
        W3 * inv_t, (b3 * inv_t).reshape(E, 1),
    )


# speedup vs baseline: 8.6984x; 3.5189x over previous
import jax
import jax.numpy as jnp
from jax import lax
from jax.experimental import pallas as pl
from jax.experimental.pallas import tpu as pltpu

N = 32768
H1 = 16
H2 = 32
E = 8


def _body(x_ref, w1_ref, b1_ref, w2_ref, b2_ref, w3_ref, b3_ref, out_ref):
    x = x_ref[...]
    h1 = jnp.maximum(w1_ref[...] * x + b1_ref[...], 0.0)
    h2 = lax.dot_general(w2_ref[...], h1, (((0,), (0,)), ((), ())),
                         preferred_element_type=jnp.float32)
    h2 = jnp.maximum(h2 + b2_ref[...], 0.0)
    lg = lax.dot_general(w3_ref[...], h2, (((0,), (0,)), ((), ())),
                         preferred_element_type=jnp.float32)
    lg = lg + b3_ref[...]
    m = jnp.max(lg, axis=0, keepdims=True)
    p = jnp.exp(lg - m)
    s = jnp.sum(p, axis=0, keepdims=True)
    out_ref[...] = p / s


def kernel(snr_estimate, temperature, W1, b1, W2, b2, W3, b3):
    inv_t = 1.0 / temperature
    outT = pl.pallas_call(
        _body,
        out_shape=jax.ShapeDtypeStruct((E, N), jnp.float32),
    )(
        snr_estimate.reshape(1, N),
        W1.reshape(H1, 1), b1.reshape(H1, 1),
        W2, b2.reshape(H2, 1),
        W3 * inv_t, (b3 * inv_t).reshape(E, 1),
    )
    return outT.T
